# Initial kernel scaffold; baseline (speedup 1.0000x reference)
import functools

import jax
import jax.numpy as jnp
from jax import lax
from jax.experimental import pallas as pl
from jax.experimental.pallas import tpu as pltpu
from jax.experimental.pallas import tpu_sc as plsc

N = 10000
E = 640000
D = 128
H = 64
T = 64

NC = 2
NS = 16
NW = NC * NS

CH = 128
NCHUNK = 157
EPW = NCHUNK * CH
EPAD = NW * EPW
ROWS_PER_TILE = N // NS

BLK = 1000
GRID = N // BLK


def _head_body(x_ref, w_ref, b_ref, o_ref):
    acc = jnp.dot(x_ref[...], w_ref[...], preferred_element_type=jnp.float32)
    o_ref[...] = jnp.maximum(acc + b_ref[...], 0.0)


def _tail_body(h_ref, a0_ref, a1_ref, w1_ref, b1_ref, w2_ref, b2_ref,
               w3_ref, b3_ref, o_ref):
    m = a0_ref[...] + a1_ref[...] - h_ref[...]
    t = jnp.dot(m, w1_ref[...], preferred_element_type=jnp.float32) + b1_ref[...]
    t = jnp.maximum(t, 0.0)
    t = jnp.dot(t, w2_ref[...], preferred_element_type=jnp.float32) + b2_ref[...]
    o_ref[...] = jnp.dot(t, w3_ref[...], preferred_element_type=jnp.float32) + b3_ref[...]


def _sc_segment_sum(src, dst, h):
    mesh = plsc.VectorSubcoreMesh(core_axis_name="c", subcore_axis_name="s")

    @functools.partial(
        pl.kernel,
        out_type=jax.ShapeDtypeStruct((NC, N, H), jnp.float32),
        mesh=mesh,
        scratch_types=[
            pltpu.VMEM_SHARED((N, H), jnp.float32),
            pltpu.VMEM_SHARED((N + 8, H), jnp.float32),
            pltpu.VMEM((CH,), jnp.int32),
            pltpu.VMEM((CH,), jnp.int32),
            pltpu.VMEM((CH, H), jnp.float32),
            pltpu.SemaphoreType.DMA,
        ],
    )
    def sc_kernel(src_hbm, dst_hbm, h_hbm, out_hbm,
                  h_sh, agg_sh, src_v, dst_v, rows_v, sem):
        c = lax.axis_index("c")
        s = lax.axis_index("s")
        wid = s * NC + c
        r0 = s * ROWS_PER_TILE
        pltpu.sync_copy(h_hbm.at[pl.ds(r0, ROWS_PER_TILE)],
                        h_sh.at[pl.ds(r0, ROWS_PER_TILE)])
        pltpu.sync_copy(h_hbm.at[pl.ds(r0, ROWS_PER_TILE)],
                        agg_sh.at[pl.ds(r0, ROWS_PER_TILE)])
        plsc.subcore_barrier()

        base = wid * EPW

        def body(k, carry):
            off = base + k * CH
            pltpu.sync_copy(src_hbm.at[pl.ds(off, CH)], src_v)
            pltpu.sync_copy(dst_hbm.at[pl.ds(off, CH)], dst_v)
            pltpu.async_copy(h_sh.at[src_v], rows_v, sem).wait()
            pltpu.sync_copy(rows_v, agg_sh.at[dst_v], add=True)
            return carry

        lax.fori_loop(0, NCHUNK, body, 0)
        plsc.subcore_barrier()
        pltpu.sync_copy(agg_sh.at[pl.ds(r0, ROWS_PER_TILE)],
                        out_hbm.at[c].at[pl.ds(r0, ROWS_PER_TILE)])

    return sc_kernel(src, dst, h)


def kernel(feature, edge_index, W0, b0, W1, b1, W2, b2, W3, b3):
    h = pl.pallas_call(
        _head_body,
        grid=(GRID,),
        in_specs=[
            pl.BlockSpec((BLK, D), lambda i: (i, 0)),
            pl.BlockSpec((D, H), lambda i: (0, 0)),
            pl.BlockSpec((1, H), lambda i: (0, 0)),
        ],
        out_specs=pl.BlockSpec((BLK, H), lambda i: (i, 0)),
        out_shape=jax.ShapeDtypeStruct((N, H), jnp.float32),
    )(feature, W0, b0.reshape(1, H))

    pad = EPAD - E
    src = jnp.concatenate([edge_index[0], jnp.zeros((pad,), jnp.int32)])
    dst = jnp.concatenate([edge_index[1], jnp.full((pad,), N, jnp.int32)])

    agg = _sc_segment_sum(src, dst, h)

    out = pl.pallas_call(
        _tail_body,
        grid=(GRID,),
        in_specs=[
            pl.BlockSpec((BLK, H), lambda i: (i, 0)),
            pl.BlockSpec((BLK, H), lambda i: (i, 0)),
            pl.BlockSpec((BLK, H), lambda i: (i, 0)),
            pl.BlockSpec((H, H), lambda i: (0, 0)),
            pl.BlockSpec((1, H), lambda i: (0, 0)),
            pl.BlockSpec((H, H), lambda i: (0, 0)),
            pl.BlockSpec((1, H), lambda i: (0, 0)),
            pl.BlockSpec((H, T), lambda i: (0, 0)),
            pl.BlockSpec((1, T), lambda i: (0, 0)),
        ],
        out_specs=pl.BlockSpec((BLK, T), lambda i: (i, 0)),
        out_shape=jax.ShapeDtypeStruct((N, T), jnp.float32),
    )(h, agg[0], agg[1], W1, b1.reshape(1, H), W2, b2.reshape(1, H),
      W3, b3.reshape(1, T))

    return out


# speedup vs baseline: 10.9880x; 10.9880x over previous
import functools

import jax
import jax.numpy as jnp
from jax import lax
from jax.experimental import pallas as pl
from jax.experimental.pallas import tpu as pltpu
from jax.experimental.pallas import tpu_sc as plsc

N = 10000
E = 640000
D = 128
H = 64
T = 64

NC = 2
NS = 16
NW = NC * NS

CH = 128
NCHUNK = 157
EPW = NCHUNK * CH
EPAD = NW * EPW
ROWS_PER_TILE = 624
ROWS_TAIL = N - NS * ROWS_PER_TILE

BLK = 1000
GRID = N // BLK


def _head_body(x_ref, w_ref, b_ref, o_ref):
    acc = jnp.dot(x_ref[...], w_ref[...], preferred_element_type=jnp.float32)
    o_ref[...] = jnp.maximum(acc + b_ref[...], 0.0)


def _tail_body(h_ref, a0_ref, a1_ref, w1_ref, b1_ref, w2_ref, b2_ref,
               w3_ref, b3_ref, o_ref):
    m = a0_ref[...] + a1_ref[...] - h_ref[...]
    t = jnp.dot(m, w1_ref[...], preferred_element_type=jnp.float32) + b1_ref[...]
    t = jnp.maximum(t, 0.0)
    t = jnp.dot(t, w2_ref[...], preferred_element_type=jnp.float32) + b2_ref[...]
    o_ref[...] = jnp.dot(t, w3_ref[...], preferred_element_type=jnp.float32) + b3_ref[...]


def _sc_segment_sum(src, dst, h):
    mesh = plsc.VectorSubcoreMesh(core_axis_name="c", subcore_axis_name="s")

    @functools.partial(
        pl.kernel,
        out_type=jax.ShapeDtypeStruct((NC, N, H), jnp.float32),
        mesh=mesh,
        compiler_params=pltpu.CompilerParams(use_tc_tiling_on_sc=False),
        scratch_types=[
            pltpu.VMEM_SHARED((N, H), jnp.float32),
            pltpu.VMEM_SHARED((N + 8, H), jnp.float32),
            pltpu.VMEM((CH,), jnp.int32),
            pltpu.VMEM((CH,), jnp.int32),
            pltpu.VMEM((CH, H), jnp.float32),
            pltpu.SemaphoreType.DMA,
        ],
    )
    def sc_kernel(src_hbm, dst_hbm, h_hbm, out_hbm,
                  h_sh, agg_sh, src_v, dst_v, rows_v, sem):
        c = lax.axis_index("c")
        s = lax.axis_index("s")
        wid = s * NC + c
        r0 = s * ROWS_PER_TILE
        pltpu.sync_copy(h_hbm.at[pl.ds(r0, ROWS_PER_TILE)],
                        h_sh.at[pl.ds(r0, ROWS_PER_TILE)])
        pltpu.sync_copy(h_hbm.at[pl.ds(r0, ROWS_PER_TILE)],
                        agg_sh.at[pl.ds(r0, ROWS_PER_TILE)])

        @pl.when(s == 0)
        def _stage_tail():
            t0 = NS * ROWS_PER_TILE
            pltpu.sync_copy(h_hbm.at[pl.ds(t0, ROWS_TAIL)],
                            h_sh.at[pl.ds(t0, ROWS_TAIL)])
            pltpu.sync_copy(h_hbm.at[pl.ds(t0, ROWS_TAIL)],
                            agg_sh.at[pl.ds(t0, ROWS_TAIL)])

        plsc.subcore_barrier()

        base = wid * EPW

        def body(k, carry):
            off = base + k * CH
            pltpu.sync_copy(src_hbm.at[pl.ds(off, CH)], src_v)
            pltpu.sync_copy(dst_hbm.at[pl.ds(off, CH)], dst_v)
            pltpu.async_copy(h_sh.at[src_v], rows_v, sem).wait()
            pltpu.sync_copy(rows_v, agg_sh.at[dst_v], add=True)
            return carry

        lax.fori_loop(0, NCHUNK, body, 0)
        plsc.subcore_barrier()
        pltpu.sync_copy(agg_sh.at[pl.ds(r0, ROWS_PER_TILE)],
                        out_hbm.at[c].at[pl.ds(r0, ROWS_PER_TILE)])

        @pl.when(s == 0)
        def _write_tail():
            t0 = NS * ROWS_PER_TILE
            pltpu.sync_copy(agg_sh.at[pl.ds(t0, ROWS_TAIL)],
                            out_hbm.at[c].at[pl.ds(t0, ROWS_TAIL)])

    return sc_kernel(src, dst, h)


def kernel(feature, edge_index, W0, b0, W1, b1, W2, b2, W3, b3):
    h = pl.pallas_call(
        _head_body,
        grid=(GRID,),
        in_specs=[
            pl.BlockSpec((BLK, D), lambda i: (i, 0)),
            pl.BlockSpec((D, H), lambda i: (0, 0)),
            pl.BlockSpec((1, H), lambda i: (0, 0)),
        ],
        out_specs=pl.BlockSpec((BLK, H), lambda i: (i, 0)),
        out_shape=jax.ShapeDtypeStruct((N, H), jnp.float32),
    )(feature, W0, b0.reshape(1, H))

    pad = EPAD - E
    src = jnp.concatenate([edge_index[0], jnp.zeros((pad,), jnp.int32)])
    dst = jnp.concatenate([edge_index[1], jnp.full((pad,), N, jnp.int32)])

    agg = _sc_segment_sum(src, dst, h)

    out = pl.pallas_call(
        _tail_body,
        grid=(GRID,),
        in_specs=[
            pl.BlockSpec((BLK, H), lambda i: (i, 0)),
            pl.BlockSpec((BLK, H), lambda i: (i, 0)),
            pl.BlockSpec((BLK, H), lambda i: (i, 0)),
            pl.BlockSpec((H, H), lambda i: (0, 0)),
            pl.BlockSpec((1, H), lambda i: (0, 0)),
            pl.BlockSpec((H, H), lambda i: (0, 0)),
            pl.BlockSpec((1, H), lambda i: (0, 0)),
            pl.BlockSpec((H, T), lambda i: (0, 0)),
            pl.BlockSpec((1, T), lambda i: (0, 0)),
        ],
        out_specs=pl.BlockSpec((BLK, T), lambda i: (i, 0)),
        out_shape=jax.ShapeDtypeStruct((N, T), jnp.float32),
    )(h, agg[0], agg[1], W1, b1.reshape(1, H), W2, b2.reshape(1, H),
      W3, b3.reshape(1, T))

    return out


# speedup vs baseline: 14.9116x; 1.3571x over previous
import functools

import jax
import jax.numpy as jnp
from jax import lax
from jax.experimental import pallas as pl
from jax.experimental.pallas import tpu as pltpu
from jax.experimental.pallas import tpu_sc as plsc

N = 10000
E = 640000
D = 128
H = 64
T = 64

NC = 2
NS = 16
NW = NC * NS

CH = 128
NBUF = 4
NCHUNK = 160
NGRP = NCHUNK // NBUF
EPW = NCHUNK * CH
EPAD = NW * EPW
ROWS_PER_TILE = 624
ROWS_TAIL = N - NS * ROWS_PER_TILE

BLK = 1000
GRID = N // BLK


def _head_body(x_ref, w_ref, b_ref, o_ref):
    acc = jnp.dot(x_ref[...], w_ref[...], preferred_element_type=jnp.float32)
    o_ref[...] = jnp.maximum(acc + b_ref[...], 0.0)


def _tail_body(h_ref, a0_ref, a1_ref, w1_ref, b1_ref, w2_ref, b2_ref,
               w3_ref, b3_ref, o_ref):
    m = a0_ref[...] + a1_ref[...] - h_ref[...]
    t = jnp.dot(m, w1_ref[...], preferred_element_type=jnp.float32) + b1_ref[...]
    t = jnp.maximum(t, 0.0)
    t = jnp.dot(t, w2_ref[...], preferred_element_type=jnp.float32) + b2_ref[...]
    o_ref[...] = jnp.dot(t, w3_ref[...], preferred_element_type=jnp.float32) + b3_ref[...]


def _sc_segment_sum(idx, h):
    mesh = plsc.VectorSubcoreMesh(core_axis_name="c", subcore_axis_name="s")

    @functools.partial(
        pl.kernel,
        out_type=jax.ShapeDtypeStruct((NC, N, H), jnp.float32),
        mesh=mesh,
        compiler_params=pltpu.CompilerParams(use_tc_tiling_on_sc=False),
        scratch_types=[
            pltpu.VMEM_SHARED((N, H), jnp.float32),
            pltpu.VMEM_SHARED((N + 8, H), jnp.float32),
            [pltpu.VMEM((2, CH), jnp.int32)] * NBUF,
            [pltpu.VMEM((CH, H), jnp.float32)] * NBUF,
            [pltpu.SemaphoreType.DMA] * NBUF,
            [pltpu.SemaphoreType.DMA] * NBUF,
            [pltpu.SemaphoreType.DMA] * NBUF,
        ],
    )
    def sc_kernel(idx_hbm, h_hbm, out_hbm,
                  h_sh, agg_sh, ibufs, rbufs, sis, sgs, sss):
        c = lax.axis_index("c")
        s = lax.axis_index("s")
        wid = s * NC + c
        r0 = s * ROWS_PER_TILE
        my_idx = idx_hbm.at[wid]

        def idx_load(j, b):
            return pltpu.async_copy(my_idx.at[j], ibufs[b], sis[b])

        def wait_idx(j, b):
            pltpu.make_async_copy(my_idx.at[j], ibufs[b], sis[b]).wait()

        def gather(j, b):
            return pltpu.async_copy(h_sh.at[ibufs[b].at[0]], rbufs[b], sgs[b])

        def wait_gather(j, b):
            pltpu.make_async_copy(h_sh.at[ibufs[b].at[0]], rbufs[b],
                                  sgs[b]).wait()

        def scatter(j, b):
            return pltpu.async_copy(rbufs[b], agg_sh.at[ibufs[b].at[1]],
                                    sss[b], add=True)

        def wait_scatter(j, b):
            pltpu.make_async_copy(rbufs[b], agg_sh.at[ibufs[b].at[1]],
                                  sss[b]).wait()

        for b in range(NBUF):
            idx_load(b, b)

        pltpu.sync_copy(h_hbm.at[pl.ds(r0, ROWS_PER_TILE)],
                        h_sh.at[pl.ds(r0, ROWS_PER_TILE)])
        pltpu.sync_copy(h_hbm.at[pl.ds(r0, ROWS_PER_TILE)],
                        agg_sh.at[pl.ds(r0, ROWS_PER_TILE)])

        @pl.when(s == 0)
        def _stage_tail():
            t0 = NS * ROWS_PER_TILE
            pltpu.sync_copy(h_hbm.at[pl.ds(t0, ROWS_TAIL)],
                            h_sh.at[pl.ds(t0, ROWS_TAIL)])
            pltpu.sync_copy(h_hbm.at[pl.ds(t0, ROWS_TAIL)],
                            agg_sh.at[pl.ds(t0, ROWS_TAIL)])

        plsc.subcore_barrier()

        def body(g, carry):
            j0 = g * NBUF
            for b in range(NBUF):
                wait_idx(j0 + b, b)
                gather(j0 + b, b)
            for b in range(NBUF):
                wait_gather(j0 + b, b)
                scatter(j0 + b, b)

            @pl.when(g + 1 < NGRP)
            def _refill():
                for b in range(NBUF):
                    wait_scatter(j0 + b, b)
                    idx_load(j0 + NBUF + b, b)

            return carry

        lax.fori_loop(0, NGRP, body, 0)
        for b in range(NBUF):
            wait_scatter((NGRP - 1) * NBUF + b, b)

        plsc.subcore_barrier()
        pltpu.sync_copy(agg_sh.at[pl.ds(r0, ROWS_PER_TILE)],
                        out_hbm.at[c].at[pl.ds(r0, ROWS_PER_TILE)])

        @pl.when(s == 0)
        def _write_tail():
            t0 = NS * ROWS_PER_TILE
            pltpu.sync_copy(agg_sh.at[pl.ds(t0, ROWS_TAIL)],
                            out_hbm.at[c].at[pl.ds(t0, ROWS_TAIL)])

    return sc_kernel(idx, h)


def kernel(feature, edge_index, W0, b0, W1, b1, W2, b2, W3, b3):
    h = pl.pallas_call(
        _head_body,
        grid=(GRID,),
        in_specs=[
            pl.BlockSpec((BLK, D), lambda i: (i, 0)),
            pl.BlockSpec((D, H), lambda i: (0, 0)),
            pl.BlockSpec((1, H), lambda i: (0, 0)),
        ],
        out_specs=pl.BlockSpec((BLK, H), lambda i: (i, 0)),
        out_shape=jax.ShapeDtypeStruct((N, H), jnp.float32),
    )(feature, W0, b0.reshape(1, H))

    pad = EPAD - E
    src = jnp.concatenate([edge_index[0], jnp.zeros((pad,), jnp.int32)])
    dst = jnp.concatenate([edge_index[1], jnp.full((pad,), N, jnp.int32)])
    idx = jnp.stack([src.reshape(NW, NCHUNK, CH),
                     dst.reshape(NW, NCHUNK, CH)], axis=2)

    agg = _sc_segment_sum(idx, h)

    out = pl.pallas_call(
        _tail_body,
        grid=(GRID,),
        in_specs=[
            pl.BlockSpec((BLK, H), lambda i: (i, 0)),
            pl.BlockSpec((BLK, H), lambda i: (i, 0)),
            pl.BlockSpec((BLK, H), lambda i: (i, 0)),
            pl.BlockSpec((H, H), lambda i: (0, 0)),
            pl.BlockSpec((1, H), lambda i: (0, 0)),
            pl.BlockSpec((H, H), lambda i: (0, 0)),
            pl.BlockSpec((1, H), lambda i: (0, 0)),
            pl.BlockSpec((H, T), lambda i: (0, 0)),
            pl.BlockSpec((1, T), lambda i: (0, 0)),
        ],
        out_specs=pl.BlockSpec((BLK, T), lambda i: (i, 0)),
        out_shape=jax.ShapeDtypeStruct((N, T), jnp.float32),
    )(h, agg[0], agg[1], W1, b1.reshape(1, H), W2, b2.reshape(1, H),
      W3, b3.reshape(1, T))

    return out


# speedup vs baseline: 16.4059x; 1.1002x over previous
import functools

import jax
import jax.numpy as jnp
from jax import lax
from jax.experimental import pallas as pl
from jax.experimental.pallas import tpu as pltpu
from jax.experimental.pallas import tpu_sc as plsc

N = 10000
E = 640000
D = 128
H = 64
T = 64

NC = 2
NS = 16
NW = NC * NS

CH = 128
NBUF = 5
NIB = 2 * NBUF
NCHUNK = 160
NGRP = NCHUNK // NBUF
EPW = NCHUNK * CH
EPAD = NW * EPW
ROWS_PER_TILE = 624
ROWS_TAIL = N - NS * ROWS_PER_TILE

BLK = 1000
GRID = N // BLK


def _head_body(x_ref, w_ref, b_ref, o_ref):
    acc = jnp.dot(x_ref[...], w_ref[...], preferred_element_type=jnp.float32)
    o_ref[...] = jnp.maximum(acc + b_ref[...], 0.0)


def _tail_body(h_ref, a0_ref, a1_ref, w1_ref, b1_ref, w2_ref, b2_ref,
               w3_ref, b3_ref, o_ref):
    m = a0_ref[...] + a1_ref[...] - h_ref[...]
    t = jnp.dot(m, w1_ref[...], preferred_element_type=jnp.float32) + b1_ref[...]
    t = jnp.maximum(t, 0.0)
    t = jnp.dot(t, w2_ref[...], preferred_element_type=jnp.float32) + b2_ref[...]
    o_ref[...] = jnp.dot(t, w3_ref[...], preferred_element_type=jnp.float32) + b3_ref[...]


def _sc_segment_sum(idx, h):
    mesh = plsc.VectorSubcoreMesh(core_axis_name="c", subcore_axis_name="s")

    @functools.partial(
        pl.kernel,
        out_type=jax.ShapeDtypeStruct((NC, N, H), jnp.float32),
        mesh=mesh,
        compiler_params=pltpu.CompilerParams(use_tc_tiling_on_sc=False),
        scratch_types=[
            pltpu.VMEM_SHARED((N, H), jnp.float32),
            pltpu.VMEM_SHARED((N + 8, H), jnp.float32),
            [pltpu.VMEM((2, CH), jnp.int32)] * NIB,
            [pltpu.VMEM((CH, H), jnp.float32)] * NBUF,
            [pltpu.SemaphoreType.DMA] * NIB,
            [pltpu.SemaphoreType.DMA] * NBUF,
            [pltpu.SemaphoreType.DMA] * NBUF,
        ],
    )
    def sc_kernel(idx_hbm, h_hbm, out_hbm,
                  h_sh, agg_sh, ibufs, rbufs, sis, sgs, sss):
        c = lax.axis_index("c")
        s = lax.axis_index("s")
        wid = s * NC + c
        r0 = s * ROWS_PER_TILE
        my_idx = idx_hbm.at[wid]

        def idx_load(j, ib):
            return pltpu.async_copy(my_idx.at[j], ibufs[ib], sis[ib])

        def wait_idx(j, ib):
            pltpu.make_async_copy(my_idx.at[j], ibufs[ib], sis[ib]).wait()

        def gather(j, ib, b):
            return pltpu.async_copy(h_sh.at[ibufs[ib].at[0]], rbufs[b], sgs[b])

        def wait_gather(j, ib, b):
            pltpu.make_async_copy(h_sh.at[ibufs[ib].at[0]], rbufs[b],
                                  sgs[b]).wait()

        def scatter(j, ib, b):
            return pltpu.async_copy(rbufs[b], agg_sh.at[ibufs[ib].at[1]],
                                    sss[b], add=True)

        def wait_scatter(j, ib, b):
            pltpu.make_async_copy(rbufs[b], agg_sh.at[ibufs[ib].at[1]],
                                  sss[b]).wait()

        for q in range(NIB):
            idx_load(q, q)

        pltpu.sync_copy(h_hbm.at[pl.ds(r0, ROWS_PER_TILE)],
                        h_sh.at[pl.ds(r0, ROWS_PER_TILE)])
        pltpu.sync_copy(h_hbm.at[pl.ds(r0, ROWS_PER_TILE)],
                        agg_sh.at[pl.ds(r0, ROWS_PER_TILE)])

        @pl.when(s == 0)
        def _stage_tail():
            t0 = NS * ROWS_PER_TILE
            pltpu.sync_copy(h_hbm.at[pl.ds(t0, ROWS_TAIL)],
                            h_sh.at[pl.ds(t0, ROWS_TAIL)])
            pltpu.sync_copy(h_hbm.at[pl.ds(t0, ROWS_TAIL)],
                            agg_sh.at[pl.ds(t0, ROWS_TAIL)])

        plsc.subcore_barrier()

        for b in range(NBUF):
            wait_idx(b, b)
            gather(b, b, b)

        def one_group(g, p):
            j0 = g * NBUF
            for b in range(NBUF):
                ib = p * NBUF + b
                wait_gather(j0 + b, ib, b)
                scatter(j0 + b, ib, b)
            for b in range(NBUF):
                ib = p * NBUF + b
                ibn = ((p + 1) % 2) * NBUF + b
                wait_scatter(j0 + b, ib, b)

                @pl.when(g + 1 < NGRP)
                def _next_gather():
                    wait_idx(j0 + NBUF + b, ibn)
                    gather(j0 + NBUF + b, ibn, b)

                @pl.when(g + 2 < NGRP)
                def _next_idx():
                    idx_load(j0 + 2 * NBUF + b, ib)

        def body(g2, carry):
            one_group(2 * g2, 0)
            one_group(2 * g2 + 1, 1)
            return carry

        lax.fori_loop(0, NGRP // 2, body, 0)

        plsc.subcore_barrier()
        pltpu.sync_copy(agg_sh.at[pl.ds(r0, ROWS_PER_TILE)],
                        out_hbm.at[c].at[pl.ds(r0, ROWS_PER_TILE)])

        @pl.when(s == 0)
        def _write_tail():
            t0 = NS * ROWS_PER_TILE
            pltpu.sync_copy(agg_sh.at[pl.ds(t0, ROWS_TAIL)],
                            out_hbm.at[c].at[pl.ds(t0, ROWS_TAIL)])

    return sc_kernel(idx, h)


def kernel(feature, edge_index, W0, b0, W1, b1, W2, b2, W3, b3):
    h = pl.pallas_call(
        _head_body,
        grid=(GRID,),
        in_specs=[
            pl.BlockSpec((BLK, D), lambda i: (i, 0)),
            pl.BlockSpec((D, H), lambda i: (0, 0)),
            pl.BlockSpec((1, H), lambda i: (0, 0)),
        ],
        out_specs=pl.BlockSpec((BLK, H), lambda i: (i, 0)),
        out_shape=jax.ShapeDtypeStruct((N, H), jnp.float32),
    )(feature, W0, b0.reshape(1, H))

    pad = EPAD - E
    src = jnp.concatenate([edge_index[0], jnp.zeros((pad,), jnp.int32)])
    dst = jnp.concatenate([edge_index[1], jnp.full((pad,), N, jnp.int32)])
    idx = jnp.stack([src.reshape(NW, NCHUNK, CH),
                     dst.reshape(NW, NCHUNK, CH)], axis=2)

    agg = _sc_segment_sum(idx, h)

    out = pl.pallas_call(
        _tail_body,
        grid=(GRID,),
        in_specs=[
            pl.BlockSpec((BLK, H), lambda i: (i, 0)),
            pl.BlockSpec((BLK, H), lambda i: (i, 0)),
            pl.BlockSpec((BLK, H), lambda i: (i, 0)),
            pl.BlockSpec((H, H), lambda i: (0, 0)),
            pl.BlockSpec((1, H), lambda i: (0, 0)),
            pl.BlockSpec((H, H), lambda i: (0, 0)),
            pl.BlockSpec((1, H), lambda i: (0, 0)),
            pl.BlockSpec((H, T), lambda i: (0, 0)),
            pl.BlockSpec((1, T), lambda i: (0, 0)),
        ],
        out_specs=pl.BlockSpec((BLK, T), lambda i: (i, 0)),
        out_shape=jax.ShapeDtypeStruct((N, T), jnp.float32),
    )(h, agg[0], agg[1], W1, b1.reshape(1, H), W2, b2.reshape(1, H),
      W3, b3.reshape(1, T))

    return out


# speedup vs baseline: 17.0918x; 1.0418x over previous
import functools

import jax
import jax.numpy as jnp
from jax import lax
from jax.experimental import pallas as pl
from jax.experimental.pallas import tpu as pltpu
from jax.experimental.pallas import tpu_sc as plsc

N = 10000
E = 640000
D = 128
H = 64
T = 64

NC = 2
NS = 16
NW = NC * NS

EPW = E // NW
CH = 128
NCHUNK = EPW // CH
CHT = EPW - NCHUNK * CH
NBUF = 4
NIB = 2 * NBUF
NGRP = NCHUNK // NBUF
ROWS_PER_TILE = 624
ROWS_TAIL = N - NS * ROWS_PER_TILE

BLK = 1000
GRID = N // BLK


def _head_body(x_ref, w_ref, b_ref, o_ref):
    acc = jnp.dot(x_ref[...], w_ref[...], preferred_element_type=jnp.float32)
    o_ref[...] = jnp.maximum(acc + b_ref[...], 0.0)


def _tail_body(h_ref, a0_ref, a1_ref, w1_ref, b1_ref, w2_ref, b2_ref,
               w3_ref, b3_ref, o_ref):
    m = a0_ref[...] + a1_ref[...] - h_ref[...]
    t = jnp.dot(m, w1_ref[...], preferred_element_type=jnp.float32) + b1_ref[...]
    t = jnp.maximum(t, 0.0)
    t = jnp.dot(t, w2_ref[...], preferred_element_type=jnp.float32) + b2_ref[...]
    o_ref[...] = jnp.dot(t, w3_ref[...], preferred_element_type=jnp.float32) + b3_ref[...]


def _sc_segment_sum(edge_index, h):
    mesh = plsc.VectorSubcoreMesh(core_axis_name="c", subcore_axis_name="s")

    @functools.partial(
        pl.kernel,
        out_type=jax.ShapeDtypeStruct((NC, N, H), jnp.float32),
        mesh=mesh,
        compiler_params=pltpu.CompilerParams(use_tc_tiling_on_sc=False),
        scratch_types=[
            pltpu.VMEM_SHARED((N, H), jnp.float32),
            pltpu.VMEM_SHARED((N + 8, H), jnp.float32),
            [pltpu.VMEM((CH,), jnp.int32)] * NIB,
            [pltpu.VMEM((CH,), jnp.int32)] * NIB,
            [pltpu.VMEM((CH, H), jnp.float32)] * NBUF,
            pltpu.VMEM((CHT,), jnp.int32),
            pltpu.VMEM((CHT,), jnp.int32),
            pltpu.VMEM((CHT, H), jnp.float32),
            [pltpu.SemaphoreType.DMA] * NIB,
            [pltpu.SemaphoreType.DMA] * NIB,
            [pltpu.SemaphoreType.DMA] * NBUF,
            [pltpu.SemaphoreType.DMA] * NBUF,
            pltpu.SemaphoreType.DMA,
        ],
    )
    def sc_kernel(edge_hbm, h_hbm, out_hbm,
                  h_sh, agg_sh, sbufs, dbufs, rbufs, sbuf_t, dbuf_t, rbuf_t,
                  sis, sid, sgs, sss, st):
        c = lax.axis_index("c")
        s = lax.axis_index("s")
        wid = s * NC + c
        base = wid * EPW
        r0 = s * ROWS_PER_TILE
        src_row = edge_hbm.at[0]
        dst_row = edge_hbm.at[1]

        def idx_load(j, ib):
            pltpu.async_copy(src_row.at[pl.ds(base + j * CH, CH)],
                             sbufs[ib], sis[ib])
            pltpu.async_copy(dst_row.at[pl.ds(base + j * CH, CH)],
                             dbufs[ib], sid[ib])

        def wait_idx(j, ib):
            pltpu.make_async_copy(src_row.at[pl.ds(base + j * CH, CH)],
                                  sbufs[ib], sis[ib]).wait()
            pltpu.make_async_copy(dst_row.at[pl.ds(base + j * CH, CH)],
                                  dbufs[ib], sid[ib]).wait()

        def gather(j, ib, b):
            pltpu.async_copy(h_sh.at[sbufs[ib]], rbufs[b], sgs[b])

        def wait_gather(j, ib, b):
            pltpu.make_async_copy(h_sh.at[sbufs[ib]], rbufs[b], sgs[b]).wait()

        def scatter(j, ib, b):
            pltpu.async_copy(rbufs[b], agg_sh.at[dbufs[ib]], sss[b], add=True)

        def wait_scatter(j, ib, b):
            pltpu.make_async_copy(rbufs[b], agg_sh.at[dbufs[ib]],
                                  sss[b]).wait()

        for q in range(NIB):
            idx_load(q, q)
        t_off = base + NCHUNK * CH
        pltpu.async_copy(src_row.at[pl.ds(t_off, CHT)], sbuf_t, st)
        pltpu.async_copy(dst_row.at[pl.ds(t_off, CHT)], dbuf_t, st)

        pltpu.sync_copy(h_hbm.at[pl.ds(r0, ROWS_PER_TILE)],
                        h_sh.at[pl.ds(r0, ROWS_PER_TILE)])
        pltpu.sync_copy(h_hbm.at[pl.ds(r0, ROWS_PER_TILE)],
                        agg_sh.at[pl.ds(r0, ROWS_PER_TILE)])

        @pl.when(s == 0)
        def _stage_tail():
            t0 = NS * ROWS_PER_TILE
            pltpu.sync_copy(h_hbm.at[pl.ds(t0, ROWS_TAIL)],
                            h_sh.at[pl.ds(t0, ROWS_TAIL)])
            pltpu.sync_copy(h_hbm.at[pl.ds(t0, ROWS_TAIL)],
                            agg_sh.at[pl.ds(t0, ROWS_TAIL)])

        plsc.subcore_barrier()

        for b in range(NBUF):
            wait_idx(b, b)
            gather(b, b, b)

        def one_group(g, p):
            j0 = g * NBUF
            for b in range(NBUF):
                ib = p * NBUF + b
                wait_gather(j0 + b, ib, b)
                scatter(j0 + b, ib, b)
            for b in range(NBUF):
                ib = p * NBUF + b
                ibn = ((p + 1) % 2) * NBUF + b
                wait_scatter(j0 + b, ib, b)

                @pl.when(g + 1 < NGRP)
                def _next_gather():
                    wait_idx(j0 + NBUF + b, ibn)
                    gather(j0 + NBUF + b, ibn, b)

                @pl.when(g + 2 < NGRP)
                def _next_idx():
                    idx_load(j0 + 2 * NBUF + b, ib)

        def body(g2, carry):
            one_group(2 * g2, 0)
            one_group(2 * g2 + 1, 1)
            return carry

        lax.fori_loop(0, NGRP // 2, body, 0)

        jp = (NGRP - 1) * NBUF
        for b in range(NBUF):
            wait_gather(jp + b, b, b)
            scatter(jp + b, b, b)
        for b in range(NBUF):
            wait_scatter(jp + b, b, b)

        pltpu.make_async_copy(src_row.at[pl.ds(t_off, CHT)], sbuf_t, st).wait()
        pltpu.make_async_copy(dst_row.at[pl.ds(t_off, CHT)], dbuf_t, st).wait()
        pltpu.async_copy(h_sh.at[sbuf_t], rbuf_t, st)
        pltpu.make_async_copy(h_sh.at[sbuf_t], rbuf_t, st).wait()
        pltpu.async_copy(rbuf_t, agg_sh.at[dbuf_t], st, add=True)
        pltpu.make_async_copy(rbuf_t, agg_sh.at[dbuf_t], st).wait()

        plsc.subcore_barrier()
        pltpu.sync_copy(agg_sh.at[pl.ds(r0, ROWS_PER_TILE)],
                        out_hbm.at[c].at[pl.ds(r0, ROWS_PER_TILE)])

        @pl.when(s == 0)
        def _write_tail():
            t0 = NS * ROWS_PER_TILE
            pltpu.sync_copy(agg_sh.at[pl.ds(t0, ROWS_TAIL)],
                            out_hbm.at[c].at[pl.ds(t0, ROWS_TAIL)])

    return sc_kernel(edge_index, h)


def kernel(feature, edge_index, W0, b0, W1, b1, W2, b2, W3, b3):
    h = pl.pallas_call(
        _head_body,
        grid=(GRID,),
        in_specs=[
            pl.BlockSpec((BLK, D), lambda i: (i, 0)),
            pl.BlockSpec((D, H), lambda i: (0, 0)),
            pl.BlockSpec((1, H), lambda i: (0, 0)),
        ],
        out_specs=pl.BlockSpec((BLK, H), lambda i: (i, 0)),
        out_shape=jax.ShapeDtypeStruct((N, H), jnp.float32),
    )(feature, W0, b0.reshape(1, H))

    agg = _sc_segment_sum(edge_index, h)

    out = pl.pallas_call(
        _tail_body,
        grid=(GRID,),
        in_specs=[
            pl.BlockSpec((BLK, H), lambda i: (i, 0)),
            pl.BlockSpec((BLK, H), lambda i: (i, 0)),
            pl.BlockSpec((BLK, H), lambda i: (i, 0)),
            pl.BlockSpec((H, H), lambda i: (0, 0)),
            pl.BlockSpec((1, H), lambda i: (0, 0)),
            pl.BlockSpec((H, H), lambda i: (0, 0)),
            pl.BlockSpec((1, H), lambda i: (0, 0)),
            pl.BlockSpec((H, T), lambda i: (0, 0)),
            pl.BlockSpec((1, T), lambda i: (0, 0)),
        ],
        out_specs=pl.BlockSpec((BLK, T), lambda i: (i, 0)),
        out_shape=jax.ShapeDtypeStruct((N, T), jnp.float32),
    )(h, agg[0], agg[1], W1, b1.reshape(1, H), W2, b2.reshape(1, H),
      W3, b3.reshape(1, T))

    return out
